# baseline (speedup 1.0000x reference)
import functools

import jax
import jax.numpy as jnp
from jax import lax
from jax.experimental import pallas as pl
from jax.experimental.pallas import tpu as pltpu
from jax.experimental.pallas import tpu_sc as plsc

N = 10000
E = 640000
D_IN = 1433
D_HID = 16
D_OUT = 7

NC = 2
NS = 16
NW = NC * NS
EB = 128
NBLK = 160
NBUF = 4
EPT = EB * NBLK
EPAD = EPT * NW
NPAD = 10240
RPT = NPAD // NS

BN = 400

_DNUMS = lax.GatherDimensionNumbers(
    offset_dims=(), collapsed_slice_dims=(0,), start_index_map=(0,))


def _bcast_lane(vec, t):
    ix = jnp.full((16, 1), t, jnp.int32)
    return lax.gather(vec, ix, _DNUMS, (1,),
                      mode=lax.GatherScatterMode.PROMISE_IN_BOUNDS)


def _sc_mesh():
    return plsc.VectorSubcoreMesh(core_axis_name="c", subcore_axis_name="s")


@functools.partial(
    pl.kernel,
    mesh=_sc_mesh(),
    compiler_params=pltpu.CompilerParams(use_tc_tiling_on_sc=False),
    out_type=[
        jax.ShapeDtypeStruct((NC, NPAD, D_HID), jnp.float32),
        jax.ShapeDtypeStruct((NC, NPAD), jnp.float32),
    ],
    scratch_types=[
        pltpu.VMEM((NBLK, EB), jnp.int32),
        pltpu.VMEM((NBLK, EB), jnp.int32),
        pltpu.VMEM((NBLK, EB), jnp.float32),
        pltpu.VMEM((EB,), jnp.float32),
        pltpu.VMEM((NBUF, EB, 2 * D_HID), jnp.float32),
        pltpu.VMEM((NBUF, EB, D_HID), jnp.float32),
        pltpu.VMEM((RPT, D_HID), jnp.float32),
        pltpu.VMEM((RPT,), jnp.float32),
        pltpu.VMEM_SHARED((NPAD, D_HID), jnp.float32),
        pltpu.VMEM_SHARED((NPAD,), jnp.float32),
        pltpu.SemaphoreType.DMA((NBUF,)),
        pltpu.SemaphoreType.DMA((NBUF,)),
        pltpu.SemaphoreType.DMA((NBUF,)),
    ],
)
def _sc_layer1(xp, srcp, dstp, up, z2d, z1d, agg_out, deg_out,
               srca, dsta, ua, onesv, rows, msg, zbuf, zvec,
               aggsh, degsh, gsem, ssem, dsem):
    c = lax.axis_index("c")
    s = lax.axis_index("s")
    wid = s * NC + c

    pltpu.sync_copy(z2d, zbuf)
    pltpu.sync_copy(z1d, zvec)
    pltpu.sync_copy(zbuf, aggsh.at[pl.ds(s * RPT, RPT)])
    pltpu.sync_copy(zvec, degsh.at[pl.ds(s * RPT, RPT)])
    for i in range(EB // 16):
        onesv[pl.ds(i * 16, 16)] = jnp.ones((16,), jnp.float32)

    pltpu.sync_copy(srcp.at[pl.ds(wid * NBLK, NBLK)], srca)
    pltpu.sync_copy(dstp.at[pl.ds(wid * NBLK, NBLK)], dsta)
    pltpu.sync_copy(up.at[pl.ds(wid * NBLK, NBLK)], ua)
    plsc.subcore_barrier()

    for b in range(NBUF):
        pltpu.async_copy(xp.at[srca.at[b]], rows.at[b], gsem.at[b])

    def pairblk(jj, carry):
        for b in range(NBUF):
            j = jj * NBUF + b
            pltpu.make_async_copy(xp.at[srca.at[b]], rows.at[b],
                                  gsem.at[b]).wait()

            @pl.when(j >= NBUF)
            def _():
                pltpu.make_async_copy(msg.at[b], aggsh.at[dsta.at[j]],
                                      ssem.at[b]).wait()
                pltpu.make_async_copy(onesv, degsh.at[dsta.at[j]],
                                      dsem.at[b]).wait()

            def group(g, carry2):
                base = g * 16
                u16 = ua[j, pl.ds(base, 16)]
                for t in range(16):
                    ub = _bcast_lane(u16, t)
                    e = base + t
                    a = rows[b, e, pl.ds(0, D_HID)]
                    bb = rows[b, e, pl.ds(D_HID, D_HID)]
                    msg[b, e, :] = a + ub * bb
                return carry2

            lax.fori_loop(0, EB // 16, group, 0)

            @pl.when(j + NBUF < NBLK)
            def _():
                pltpu.async_copy(xp.at[srca.at[j + NBUF]], rows.at[b],
                                 gsem.at[b])

            pltpu.async_copy(msg.at[b], aggsh.at[dsta.at[j]], ssem.at[b],
                             add=True)
            pltpu.async_copy(onesv, degsh.at[dsta.at[j]], dsem.at[b],
                             add=True)
        return carry

    lax.fori_loop(0, NBLK // NBUF, pairblk, 0)
    for b in range(NBUF):
        pltpu.make_async_copy(msg.at[b], aggsh.at[dsta.at[b]],
                              ssem.at[b]).wait()
        pltpu.make_async_copy(onesv, degsh.at[dsta.at[b]],
                              dsem.at[b]).wait()
    plsc.subcore_barrier()

    pltpu.sync_copy(aggsh.at[pl.ds(s * RPT, RPT)],
                    agg_out.at[c, pl.ds(s * RPT, RPT)])
    pltpu.sync_copy(degsh.at[pl.ds(s * RPT, RPT)],
                    deg_out.at[c, pl.ds(s * RPT, RPT)])


@functools.partial(
    pl.kernel,
    mesh=_sc_mesh(),
    compiler_params=pltpu.CompilerParams(use_tc_tiling_on_sc=False),
    out_type=jax.ShapeDtypeStruct((NC, NPAD, D_HID), jnp.float32),
    scratch_types=[
        pltpu.VMEM((NBLK, EB), jnp.int32),
        pltpu.VMEM((NBLK, EB), jnp.int32),
        pltpu.VMEM((NBLK, EB), jnp.float32),
        pltpu.VMEM((NBUF, EB, D_HID), jnp.float32),
        pltpu.VMEM((NBUF, EB, D_HID), jnp.float32),
        pltpu.VMEM((RPT, D_HID), jnp.float32),
        pltpu.VMEM_SHARED((NPAD, D_HID), jnp.float32),
        pltpu.SemaphoreType.DMA((NBUF,)),
        pltpu.SemaphoreType.DMA((NBUF,)),
    ],
)
def _sc_layer2(hp, srcp, dstp, up, z2d, agg_out,
               srca, dsta, ua, rows, msg, zbuf, aggsh, gsem, ssem):
    c = lax.axis_index("c")
    s = lax.axis_index("s")
    wid = s * NC + c

    pltpu.sync_copy(z2d, zbuf)
    pltpu.sync_copy(zbuf, aggsh.at[pl.ds(s * RPT, RPT)])
    pltpu.sync_copy(srcp.at[pl.ds(wid * NBLK, NBLK)], srca)
    pltpu.sync_copy(dstp.at[pl.ds(wid * NBLK, NBLK)], dsta)
    pltpu.sync_copy(up.at[pl.ds(wid * NBLK, NBLK)], ua)
    plsc.subcore_barrier()

    lanes = lax.iota(jnp.int32, 16)
    hi_sel = lanes < 8
    shift_ix = jnp.bitwise_or(lanes, 8).reshape(16, 1)

    for b in range(NBUF):
        pltpu.async_copy(hp.at[srca.at[b]], rows.at[b], gsem.at[b])

    def pairblk(jj, carry):
        for b in range(NBUF):
            j = jj * NBUF + b
            pltpu.make_async_copy(hp.at[srca.at[b]], rows.at[b],
                                  gsem.at[b]).wait()

            @pl.when(j >= NBUF)
            def _():
                pltpu.make_async_copy(msg.at[b], aggsh.at[dsta.at[j]],
                                      ssem.at[b]).wait()

            def group(g, carry2):
                base = g * 16
                u16 = ua[j, pl.ds(base, 16)]
                for t in range(16):
                    ub = _bcast_lane(u16, t)
                    e = base + t
                    v = rows[b, e, :]
                    w = v * jnp.where(hi_sel, jnp.float32(1.0), ub)
                    msg[b, e, :] = w + lax.gather(
                        w, shift_ix, _DNUMS, (1,),
                        mode=lax.GatherScatterMode.PROMISE_IN_BOUNDS)
                return carry2

            lax.fori_loop(0, EB // 16, group, 0)

            @pl.when(j + NBUF < NBLK)
            def _():
                pltpu.async_copy(hp.at[srca.at[j + NBUF]], rows.at[b],
                                 gsem.at[b])

            pltpu.async_copy(msg.at[b], aggsh.at[dsta.at[j]], ssem.at[b],
                             add=True)
        return carry

    lax.fori_loop(0, NBLK // NBUF, pairblk, 0)
    for b in range(NBUF):
        pltpu.make_async_copy(msg.at[b], aggsh.at[dsta.at[b]],
                              ssem.at[b]).wait()
    plsc.subcore_barrier()

    pltpu.sync_copy(aggsh.at[pl.ds(s * RPT, RPT)],
                    agg_out.at[c, pl.ds(s * RPT, RPT)])


BNT = 1280


def _tc_pack1_body(wt_ref, xt_ref, xpg_ref, xr_ref):
    y = jnp.dot(wt_ref[...], xt_ref[...], preferred_element_type=jnp.float32)
    xpg_ref[...] = y[: 2 * D_HID, :].T
    xr_ref[...] = y[2 * D_HID:, :].T


def _tc_mid_body(a0, a1, d0, d1, xr, b1r, w2, hp_ref, hr_ref):
    deg = jnp.maximum(d0[...] + d1[...], 1.0)
    h = (a0[...] + a1[...]) / deg + xr[...] + b1r[...]
    h = jnp.where(h > 0, h, jnp.exp(h) - 1.0)
    y = jnp.dot(h, w2[...], preferred_element_type=jnp.float32)
    hp_ref[...] = y[:, :D_HID]
    hr_ref[...] = y[:, D_HID:D_HID + D_OUT]


def _tc_out_body(b0, b1, d0, d1, hr, b2r, o_ref):
    deg = jnp.maximum(d0[...] + d1[...], 1.0)
    sc = (b0[...] + b1[...])[:, :D_OUT] / deg + hr[...] + b2r[...]
    m = jnp.max(sc, axis=1, keepdims=True)
    ex = jnp.exp(sc - m)
    o_ref[...] = (sc - m) - jnp.log(jnp.sum(ex, axis=1, keepdims=True))


def kernel(x, edge_index, pseudo, W1, root1, bias1, W2, root2, bias2):
    f32 = jnp.float32
    wcat = jnp.concatenate([W1[0], W1[1] - W1[0], root1], axis=1)
    w2all = jnp.zeros((D_HID, 24), f32)
    w2all = (w2all.at[:, 0:D_OUT].set(W2[0])
                  .at[:, 8:8 + D_OUT].set(W2[1] - W2[0])
                  .at[:, 16:16 + D_OUT].set(root2))

    src = edge_index[0]
    dst = edge_index[1]
    u = pseudo[:, 0]
    pad = EPAD - E
    srcp = jnp.concatenate([src, jnp.zeros((pad,), jnp.int32)]).reshape(
        NW * NBLK, EB)
    dstp = jnp.concatenate([dst, jnp.full((pad,), N, jnp.int32)]).reshape(
        NW * NBLK, EB)
    up = jnp.concatenate([u, jnp.zeros((pad,), f32)]).reshape(NW * NBLK, EB)
    z2d = jnp.zeros((RPT, D_HID), f32)
    z1d = jnp.zeros((RPT,), f32)

    xpg, xr = pl.pallas_call(
        _tc_pack1_body,
        grid=((N + BNT - 1) // BNT,),
        in_specs=[pl.BlockSpec((48, D_IN), lambda i: (0, 0)),
                  pl.BlockSpec((D_IN, BNT), lambda i: (0, i))],
        out_specs=[pl.BlockSpec((BNT, 2 * D_HID), lambda i: (i, 0)),
                   pl.BlockSpec((BNT, D_HID), lambda i: (i, 0))],
        out_shape=[jax.ShapeDtypeStruct((N, 2 * D_HID), f32),
                   jax.ShapeDtypeStruct((N, D_HID), f32)],
    )(wcat.T, x.T)

    agg1, deg = _sc_layer1(xpg, srcp, dstp, up, z2d, z1d)
    d0 = deg[0].reshape(NPAD, 1)
    d1 = deg[1].reshape(NPAD, 1)

    hp, hr = pl.pallas_call(
        _tc_mid_body,
        grid=(N // BN,),
        in_specs=[pl.BlockSpec((BN, D_HID), lambda i: (i, 0)),
                  pl.BlockSpec((BN, D_HID), lambda i: (i, 0)),
                  pl.BlockSpec((BN, 1), lambda i: (i, 0)),
                  pl.BlockSpec((BN, 1), lambda i: (i, 0)),
                  pl.BlockSpec((BN, D_HID), lambda i: (i, 0)),
                  pl.BlockSpec((1, D_HID), lambda i: (0, 0)),
                  pl.BlockSpec((D_HID, 24), lambda i: (0, 0))],
        out_specs=[pl.BlockSpec((BN, D_HID), lambda i: (i, 0)),
                   pl.BlockSpec((BN, D_OUT), lambda i: (i, 0))],
        out_shape=[jax.ShapeDtypeStruct((N, D_HID), f32),
                   jax.ShapeDtypeStruct((N, D_OUT), f32)],
    )(agg1[0], agg1[1], d0, d1, xr, bias1.reshape(1, D_HID), w2all)

    agg2 = _sc_layer2(hp, srcp, dstp, up, z2d)

    out = pl.pallas_call(
        _tc_out_body,
        grid=(N // BN,),
        in_specs=[pl.BlockSpec((BN, D_HID), lambda i: (i, 0)),
                  pl.BlockSpec((BN, D_HID), lambda i: (i, 0)),
                  pl.BlockSpec((BN, 1), lambda i: (i, 0)),
                  pl.BlockSpec((BN, 1), lambda i: (i, 0)),
                  pl.BlockSpec((BN, D_OUT), lambda i: (i, 0)),
                  pl.BlockSpec((1, D_OUT), lambda i: (0, 0))],
        out_specs=pl.BlockSpec((BN, D_OUT), lambda i: (i, 0)),
        out_shape=jax.ShapeDtypeStruct((N, D_OUT), f32),
    )(agg2[0], agg2[1], d0, d1, hr, bias2.reshape(1, D_OUT))
    return out


# speedup vs baseline: 1.2335x; 1.2335x over previous
import functools

import jax
import jax.numpy as jnp
from jax import lax
from jax.experimental import pallas as pl
from jax.experimental.pallas import tpu as pltpu
from jax.experimental.pallas import tpu_sc as plsc

N = 10000
E = 640000
D_IN = 1433
D_HID = 16
D_OUT = 7

NC = 2
NS = 16
NW = NC * NS
EB = 128
NBLK = 158
NBUF = 2
EPT = EB * NBLK
EPAD = EPT * NW
NPAD = 10240
RPT = NPAD // NS

BN = 400

_DNUMS = lax.GatherDimensionNumbers(
    offset_dims=(), collapsed_slice_dims=(0,), start_index_map=(0,))


def _bcast_lane(vec, t):
    ix = jnp.full((16, 1), t, jnp.int32)
    return lax.gather(vec, ix, _DNUMS, (1,),
                      mode=lax.GatherScatterMode.PROMISE_IN_BOUNDS)


def _sc_mesh():
    return plsc.VectorSubcoreMesh(core_axis_name="c", subcore_axis_name="s")


@functools.partial(
    pl.kernel,
    mesh=_sc_mesh(),
    compiler_params=pltpu.CompilerParams(use_tc_tiling_on_sc=False),
    out_type=[
        jax.ShapeDtypeStruct((NC, NPAD, D_HID), jnp.float32),
        jax.ShapeDtypeStruct((NC, NPAD), jnp.float32),
    ],
    scratch_types=[
        pltpu.VMEM((NBLK, EB), jnp.int32),
        pltpu.VMEM((NBLK, EB), jnp.int32),
        pltpu.VMEM((NBLK, EB), jnp.float32),
        pltpu.VMEM((EB,), jnp.float32),
        pltpu.VMEM((NBUF, EB, 2 * D_HID), jnp.float32),
        pltpu.VMEM((NBUF, EB, D_HID), jnp.float32),
        pltpu.VMEM((RPT, D_HID), jnp.float32),
        pltpu.VMEM((RPT,), jnp.float32),
        pltpu.VMEM_SHARED((NPAD, D_HID), jnp.float32),
        pltpu.VMEM_SHARED((NPAD,), jnp.float32),
        pltpu.SemaphoreType.DMA((NBUF,)),
        pltpu.SemaphoreType.DMA((NBUF,)),
        pltpu.SemaphoreType.DMA((NBUF,)),
    ],
)
def _sc_layer1(xp, srcp, dstp, up, z2d, z1d, agg_out, deg_out,
               srca, dsta, ua, onesv, rows, msg, zbuf, zvec,
               aggsh, degsh, gsem, ssem, dsem):
    c = lax.axis_index("c")
    s = lax.axis_index("s")
    wid = s * NC + c

    pltpu.sync_copy(z2d, zbuf)
    pltpu.sync_copy(z1d, zvec)
    pltpu.sync_copy(zbuf, aggsh.at[pl.ds(s * RPT, RPT)])
    pltpu.sync_copy(zvec, degsh.at[pl.ds(s * RPT, RPT)])
    for i in range(EB // 16):
        onesv[pl.ds(i * 16, 16)] = jnp.ones((16,), jnp.float32)

    pltpu.sync_copy(srcp.at[pl.ds(wid * NBLK, NBLK)], srca)
    pltpu.sync_copy(dstp.at[pl.ds(wid * NBLK, NBLK)], dsta)
    pltpu.sync_copy(up.at[pl.ds(wid * NBLK, NBLK)], ua)
    plsc.subcore_barrier()

    for b in range(NBUF):
        pltpu.async_copy(xp.at[srca.at[b]], rows.at[b], gsem.at[b])

    def pairblk(jj, carry):
        for b in range(NBUF):
            j = jj * NBUF + b
            pltpu.make_async_copy(xp.at[srca.at[b]], rows.at[b],
                                  gsem.at[b]).wait()

            @pl.when(j >= NBUF)
            def _():
                pltpu.make_async_copy(msg.at[b], aggsh.at[dsta.at[j]],
                                      ssem.at[b]).wait()
                pltpu.make_async_copy(onesv, degsh.at[dsta.at[j]],
                                      dsem.at[b]).wait()

            def group(g, carry2):
                base = g * 16
                u16 = ua[j, pl.ds(base, 16)]
                for t in range(16):
                    ub = _bcast_lane(u16, t)
                    e = base + t
                    a = rows[b, e, pl.ds(0, D_HID)]
                    bb = rows[b, e, pl.ds(D_HID, D_HID)]
                    msg[b, e, :] = a + ub * bb
                return carry2

            lax.fori_loop(0, EB // 16, group, 0)

            @pl.when(j + NBUF < NBLK)
            def _():
                pltpu.async_copy(xp.at[srca.at[j + NBUF]], rows.at[b],
                                 gsem.at[b])

            pltpu.async_copy(msg.at[b], aggsh.at[dsta.at[j]], ssem.at[b],
                             add=True)
            pltpu.async_copy(onesv, degsh.at[dsta.at[j]], dsem.at[b],
                             add=True)
        return carry

    lax.fori_loop(0, NBLK // NBUF, pairblk, 0)
    for b in range(NBUF):
        pltpu.make_async_copy(msg.at[b], aggsh.at[dsta.at[b]],
                              ssem.at[b]).wait()
        pltpu.make_async_copy(onesv, degsh.at[dsta.at[b]],
                              dsem.at[b]).wait()
    plsc.subcore_barrier()

    pltpu.sync_copy(aggsh.at[pl.ds(s * RPT, RPT)],
                    agg_out.at[c, pl.ds(s * RPT, RPT)])
    pltpu.sync_copy(degsh.at[pl.ds(s * RPT, RPT)],
                    deg_out.at[c, pl.ds(s * RPT, RPT)])


@functools.partial(
    pl.kernel,
    mesh=_sc_mesh(),
    compiler_params=pltpu.CompilerParams(use_tc_tiling_on_sc=False),
    out_type=jax.ShapeDtypeStruct((NC, NPAD, D_HID), jnp.float32),
    scratch_types=[
        pltpu.VMEM((NBLK, EB), jnp.int32),
        pltpu.VMEM((NBLK, EB), jnp.int32),
        pltpu.VMEM((NBLK, EB), jnp.float32),
        pltpu.VMEM((NBUF, EB, D_HID), jnp.float32),
        pltpu.VMEM((NBUF, EB, D_HID), jnp.float32),
        pltpu.VMEM((RPT, D_HID), jnp.float32),
        pltpu.VMEM_SHARED((NPAD, D_HID), jnp.float32),
        pltpu.SemaphoreType.DMA((NBUF,)),
        pltpu.SemaphoreType.DMA((NBUF,)),
    ],
)
def _sc_layer2(hp, srcp, dstp, up, z2d, agg_out,
               srca, dsta, ua, rows, msg, zbuf, aggsh, gsem, ssem):
    c = lax.axis_index("c")
    s = lax.axis_index("s")
    wid = s * NC + c

    pltpu.sync_copy(z2d, zbuf)
    pltpu.sync_copy(zbuf, aggsh.at[pl.ds(s * RPT, RPT)])
    pltpu.sync_copy(srcp.at[pl.ds(wid * NBLK, NBLK)], srca)
    pltpu.sync_copy(dstp.at[pl.ds(wid * NBLK, NBLK)], dsta)
    pltpu.sync_copy(up.at[pl.ds(wid * NBLK, NBLK)], ua)
    plsc.subcore_barrier()

    lanes = lax.iota(jnp.int32, 16)
    hi_sel = lanes < 8
    shift_ix = jnp.bitwise_or(lanes, 8).reshape(16, 1)

    for b in range(NBUF):
        pltpu.async_copy(hp.at[srca.at[b]], rows.at[b], gsem.at[b])

    def pairblk(jj, carry):
        for b in range(NBUF):
            j = jj * NBUF + b
            pltpu.make_async_copy(hp.at[srca.at[b]], rows.at[b],
                                  gsem.at[b]).wait()

            @pl.when(j >= NBUF)
            def _():
                pltpu.make_async_copy(msg.at[b], aggsh.at[dsta.at[j]],
                                      ssem.at[b]).wait()

            def group(g, carry2):
                base = g * 16
                u16 = ua[j, pl.ds(base, 16)]
                for t in range(16):
                    ub = _bcast_lane(u16, t)
                    e = base + t
                    v = rows[b, e, :]
                    w = v * jnp.where(hi_sel, jnp.float32(1.0), ub)
                    msg[b, e, :] = w + lax.gather(
                        w, shift_ix, _DNUMS, (1,),
                        mode=lax.GatherScatterMode.PROMISE_IN_BOUNDS)
                return carry2

            lax.fori_loop(0, EB // 16, group, 0)

            @pl.when(j + NBUF < NBLK)
            def _():
                pltpu.async_copy(hp.at[srca.at[j + NBUF]], rows.at[b],
                                 gsem.at[b])

            pltpu.async_copy(msg.at[b], aggsh.at[dsta.at[j]], ssem.at[b],
                             add=True)
        return carry

    lax.fori_loop(0, NBLK // NBUF, pairblk, 0)
    for b in range(NBUF):
        pltpu.make_async_copy(msg.at[b], aggsh.at[dsta.at[b]],
                              ssem.at[b]).wait()
    plsc.subcore_barrier()

    pltpu.sync_copy(aggsh.at[pl.ds(s * RPT, RPT)],
                    agg_out.at[c, pl.ds(s * RPT, RPT)])


BNT = 1280


def _tc_pack1_body(wt_ref, xt_ref, xpg_ref, xr_ref):
    y = jnp.dot(wt_ref[...], xt_ref[...], preferred_element_type=jnp.float32)
    xpg_ref[...] = y[: 2 * D_HID, :].T
    xr_ref[...] = y[2 * D_HID:, :].T


def _tc_mid_body(a0, a1, d0, d1, xr, b1r, w2, hp_ref, hr_ref):
    deg = jnp.maximum(d0[...] + d1[...], 1.0)
    h = (a0[...] + a1[...]) / deg + xr[...] + b1r[...]
    h = jnp.where(h > 0, h, jnp.exp(h) - 1.0)
    y = jnp.dot(h, w2[...], preferred_element_type=jnp.float32)
    hp_ref[...] = y[:, :D_HID]
    hr_ref[...] = y[:, D_HID:D_HID + D_OUT]


def _tc_out_body(b0, b1, d0, d1, hr, b2r, o_ref):
    deg = jnp.maximum(d0[...] + d1[...], 1.0)
    sc = (b0[...] + b1[...])[:, :D_OUT] / deg + hr[...] + b2r[...]
    m = jnp.max(sc, axis=1, keepdims=True)
    ex = jnp.exp(sc - m)
    o_ref[...] = (sc - m) - jnp.log(jnp.sum(ex, axis=1, keepdims=True))


def kernel(x, edge_index, pseudo, W1, root1, bias1, W2, root2, bias2):
    f32 = jnp.float32
    wcat = jnp.concatenate([W1[0], W1[1] - W1[0], root1], axis=1)
    w2all = jnp.zeros((D_HID, 24), f32)
    w2all = (w2all.at[:, 0:D_OUT].set(W2[0])
                  .at[:, 8:8 + D_OUT].set(W2[1] - W2[0])
                  .at[:, 16:16 + D_OUT].set(root2))

    src = edge_index[0]
    dst = edge_index[1]
    u = pseudo[:, 0]
    pad = EPAD - E
    srcp = jnp.concatenate([src, jnp.zeros((pad,), jnp.int32)]).reshape(
        NW * NBLK, EB)
    dstp = jnp.concatenate([dst, jnp.full((pad,), N, jnp.int32)]).reshape(
        NW * NBLK, EB)
    up = jnp.concatenate([u, jnp.zeros((pad,), f32)]).reshape(NW * NBLK, EB)
    z2d = jnp.zeros((RPT, D_HID), f32)
    z1d = jnp.zeros((RPT,), f32)

    xpg, xr = pl.pallas_call(
        _tc_pack1_body,
        grid=((N + BNT - 1) // BNT,),
        in_specs=[pl.BlockSpec((48, D_IN), lambda i: (0, 0)),
                  pl.BlockSpec((D_IN, BNT), lambda i: (0, i))],
        out_specs=[pl.BlockSpec((BNT, 2 * D_HID), lambda i: (i, 0)),
                   pl.BlockSpec((BNT, D_HID), lambda i: (i, 0))],
        out_shape=[jax.ShapeDtypeStruct((N, 2 * D_HID), f32),
                   jax.ShapeDtypeStruct((N, D_HID), f32)],
    )(wcat.T, x.T)

    agg1, deg = _sc_layer1(xpg, srcp, dstp, up, z2d, z1d)
    d0 = deg[0].reshape(NPAD, 1)
    d1 = deg[1].reshape(NPAD, 1)

    hp, hr = pl.pallas_call(
        _tc_mid_body,
        grid=(N // BN,),
        in_specs=[pl.BlockSpec((BN, D_HID), lambda i: (i, 0)),
                  pl.BlockSpec((BN, D_HID), lambda i: (i, 0)),
                  pl.BlockSpec((BN, 1), lambda i: (i, 0)),
                  pl.BlockSpec((BN, 1), lambda i: (i, 0)),
                  pl.BlockSpec((BN, D_HID), lambda i: (i, 0)),
                  pl.BlockSpec((1, D_HID), lambda i: (0, 0)),
                  pl.BlockSpec((D_HID, 24), lambda i: (0, 0))],
        out_specs=[pl.BlockSpec((BN, D_HID), lambda i: (i, 0)),
                   pl.BlockSpec((BN, D_OUT), lambda i: (i, 0))],
        out_shape=[jax.ShapeDtypeStruct((N, D_HID), f32),
                   jax.ShapeDtypeStruct((N, D_OUT), f32)],
    )(agg1[0], agg1[1], d0, d1, xr, bias1.reshape(1, D_HID), w2all)

    agg2 = _sc_layer2(hp, srcp, dstp, up, z2d)

    out = pl.pallas_call(
        _tc_out_body,
        grid=(N // BN,),
        in_specs=[pl.BlockSpec((BN, D_HID), lambda i: (i, 0)),
                  pl.BlockSpec((BN, D_HID), lambda i: (i, 0)),
                  pl.BlockSpec((BN, 1), lambda i: (i, 0)),
                  pl.BlockSpec((BN, 1), lambda i: (i, 0)),
                  pl.BlockSpec((BN, D_OUT), lambda i: (i, 0)),
                  pl.BlockSpec((1, D_OUT), lambda i: (0, 0))],
        out_specs=pl.BlockSpec((BN, D_OUT), lambda i: (i, 0)),
        out_shape=jax.ShapeDtypeStruct((N, D_OUT), f32),
    )(agg2[0], agg2[1], d0, d1, hr, bias2.reshape(1, D_OUT))
    return out


# speedup vs baseline: 1.3210x; 1.0709x over previous
import functools

import jax
import jax.numpy as jnp
from jax import lax
from jax.experimental import pallas as pl
from jax.experimental.pallas import tpu as pltpu
from jax.experimental.pallas import tpu_sc as plsc

N = 10000
E = 640000
D_IN = 1433
D_HID = 16
D_OUT = 7

NC = 2
NS = 16
NW = NC * NS
EB = 128
NBLK = 158
NBUF = 2
EPT = EB * NBLK
EPAD = EPT * NW
NPAD = 10240
RPT = NPAD // NS

BN = 400

_DNUMS = lax.GatherDimensionNumbers(
    offset_dims=(), collapsed_slice_dims=(0,), start_index_map=(0,))


def _bcast_lane(vec, t):
    ix = jnp.full((16, 1), t, jnp.int32)
    return lax.gather(vec, ix, _DNUMS, (1,),
                      mode=lax.GatherScatterMode.PROMISE_IN_BOUNDS)


def _sc_mesh():
    return plsc.VectorSubcoreMesh(core_axis_name="c", subcore_axis_name="s")


@functools.partial(
    pl.kernel,
    mesh=_sc_mesh(),
    compiler_params=pltpu.CompilerParams(use_tc_tiling_on_sc=False),
    out_type=[
        jax.ShapeDtypeStruct((NC, NPAD, D_HID), jnp.float32),
        jax.ShapeDtypeStruct((NC, NPAD), jnp.float32),
    ],
    scratch_types=[
        pltpu.VMEM((NBLK, EB), jnp.int32),
        pltpu.VMEM((NBLK, EB), jnp.int32),
        pltpu.VMEM((NBLK, EB), jnp.float32),
        pltpu.VMEM((EB,), jnp.float32),
        pltpu.VMEM((NBUF, EB, 2 * D_HID), jnp.float32),
        pltpu.VMEM((NBUF, EB, D_HID), jnp.float32),
        pltpu.VMEM((RPT, D_HID), jnp.float32),
        pltpu.VMEM((RPT,), jnp.float32),
        pltpu.VMEM_SHARED((NPAD, D_HID), jnp.float32),
        pltpu.VMEM_SHARED((NPAD,), jnp.float32),
        pltpu.SemaphoreType.DMA((NBUF,)),
        pltpu.SemaphoreType.DMA((NBUF,)),
        pltpu.SemaphoreType.DMA((NBUF,)),
    ],
)
def _sc_layer1(xp, srcp, dstp, up, z2d, z1d, agg_out, deg_out,
               srca, dsta, ua, onesv, rows, msg, zbuf, zvec,
               aggsh, degsh, gsem, ssem, dsem):
    c = lax.axis_index("c")
    s = lax.axis_index("s")
    wid = s * NC + c

    pltpu.sync_copy(z2d, zbuf)
    pltpu.sync_copy(z1d, zvec)
    pltpu.sync_copy(zbuf, aggsh.at[pl.ds(s * RPT, RPT)])
    pltpu.sync_copy(zvec, degsh.at[pl.ds(s * RPT, RPT)])
    for i in range(EB // 16):
        onesv[pl.ds(i * 16, 16)] = jnp.ones((16,), jnp.float32)

    pltpu.sync_copy(srcp.at[pl.ds(wid * NBLK, NBLK)], srca)
    pltpu.sync_copy(dstp.at[pl.ds(wid * NBLK, NBLK)], dsta)
    pltpu.sync_copy(up.at[pl.ds(wid * NBLK, NBLK)], ua)
    plsc.subcore_barrier()

    for b in range(NBUF):
        pltpu.async_copy(xp.at[srca.at[b]], rows.at[b], gsem.at[b])

    def pairblk(jj, carry):
        for b in range(NBUF):
            j = jj * NBUF + b
            pltpu.make_async_copy(xp.at[srca.at[b]], rows.at[b],
                                  gsem.at[b]).wait()

            @pl.when(j >= NBUF)
            def _():
                pltpu.make_async_copy(msg.at[b], aggsh.at[dsta.at[j]],
                                      ssem.at[b]).wait()
                pltpu.make_async_copy(onesv, degsh.at[dsta.at[j]],
                                      dsem.at[b]).wait()

            for g in range(EB // 16):
                base = g * 16
                u16 = ua[j, pl.ds(base, 16)]
                for t in range(16):
                    ub = _bcast_lane(u16, t)
                    e = base + t
                    a = rows[b, e, pl.ds(0, D_HID)]
                    bb = rows[b, e, pl.ds(D_HID, D_HID)]
                    msg[b, e, :] = a + ub * bb

            @pl.when(j + NBUF < NBLK)
            def _():
                pltpu.async_copy(xp.at[srca.at[j + NBUF]], rows.at[b],
                                 gsem.at[b])

            pltpu.async_copy(msg.at[b], aggsh.at[dsta.at[j]], ssem.at[b],
                             add=True)
            pltpu.async_copy(onesv, degsh.at[dsta.at[j]], dsem.at[b],
                             add=True)
        return carry

    lax.fori_loop(0, NBLK // NBUF, pairblk, 0)
    for b in range(NBUF):
        pltpu.make_async_copy(msg.at[b], aggsh.at[dsta.at[b]],
                              ssem.at[b]).wait()
        pltpu.make_async_copy(onesv, degsh.at[dsta.at[b]],
                              dsem.at[b]).wait()
    plsc.subcore_barrier()

    pltpu.sync_copy(aggsh.at[pl.ds(s * RPT, RPT)],
                    agg_out.at[c, pl.ds(s * RPT, RPT)])
    pltpu.sync_copy(degsh.at[pl.ds(s * RPT, RPT)],
                    deg_out.at[c, pl.ds(s * RPT, RPT)])


@functools.partial(
    pl.kernel,
    mesh=_sc_mesh(),
    compiler_params=pltpu.CompilerParams(use_tc_tiling_on_sc=False),
    out_type=jax.ShapeDtypeStruct((NC, NPAD, D_HID), jnp.float32),
    scratch_types=[
        pltpu.VMEM((NBLK, EB), jnp.int32),
        pltpu.VMEM((NBLK, EB), jnp.int32),
        pltpu.VMEM((NBLK, EB), jnp.float32),
        pltpu.VMEM((NBUF, EB, D_HID), jnp.float32),
        pltpu.VMEM((NBUF, EB, D_HID), jnp.float32),
        pltpu.VMEM((RPT, D_HID), jnp.float32),
        pltpu.VMEM_SHARED((NPAD, D_HID), jnp.float32),
        pltpu.SemaphoreType.DMA((NBUF,)),
        pltpu.SemaphoreType.DMA((NBUF,)),
    ],
)
def _sc_layer2(hp, srcp, dstp, up, z2d, agg_out,
               srca, dsta, ua, rows, msg, zbuf, aggsh, gsem, ssem):
    c = lax.axis_index("c")
    s = lax.axis_index("s")
    wid = s * NC + c

    pltpu.sync_copy(z2d, zbuf)
    pltpu.sync_copy(zbuf, aggsh.at[pl.ds(s * RPT, RPT)])
    pltpu.sync_copy(srcp.at[pl.ds(wid * NBLK, NBLK)], srca)
    pltpu.sync_copy(dstp.at[pl.ds(wid * NBLK, NBLK)], dsta)
    pltpu.sync_copy(up.at[pl.ds(wid * NBLK, NBLK)], ua)
    plsc.subcore_barrier()

    lanes = lax.iota(jnp.int32, 16)
    hi_sel = lanes < 8
    shift_ix = jnp.bitwise_or(lanes, 8).reshape(16, 1)

    for b in range(NBUF):
        pltpu.async_copy(hp.at[srca.at[b]], rows.at[b], gsem.at[b])

    def pairblk(jj, carry):
        for b in range(NBUF):
            j = jj * NBUF + b
            pltpu.make_async_copy(hp.at[srca.at[b]], rows.at[b],
                                  gsem.at[b]).wait()

            @pl.when(j >= NBUF)
            def _():
                pltpu.make_async_copy(msg.at[b], aggsh.at[dsta.at[j]],
                                      ssem.at[b]).wait()

            for g in range(EB // 16):
                base = g * 16
                u16 = ua[j, pl.ds(base, 16)]
                for t in range(16):
                    ub = _bcast_lane(u16, t)
                    e = base + t
                    v = rows[b, e, :]
                    w = v * jnp.where(hi_sel, jnp.float32(1.0), ub)
                    msg[b, e, :] = w + lax.gather(
                        w, shift_ix, _DNUMS, (1,),
                        mode=lax.GatherScatterMode.PROMISE_IN_BOUNDS)

            @pl.when(j + NBUF < NBLK)
            def _():
                pltpu.async_copy(hp.at[srca.at[j + NBUF]], rows.at[b],
                                 gsem.at[b])

            pltpu.async_copy(msg.at[b], aggsh.at[dsta.at[j]], ssem.at[b],
                             add=True)
        return carry

    lax.fori_loop(0, NBLK // NBUF, pairblk, 0)
    for b in range(NBUF):
        pltpu.make_async_copy(msg.at[b], aggsh.at[dsta.at[b]],
                              ssem.at[b]).wait()
    plsc.subcore_barrier()

    pltpu.sync_copy(aggsh.at[pl.ds(s * RPT, RPT)],
                    agg_out.at[c, pl.ds(s * RPT, RPT)])


BNT = 1280


def _tc_pack1_body(wt_ref, xt_ref, xpg_ref, xr_ref):
    y = jnp.dot(wt_ref[...], xt_ref[...], preferred_element_type=jnp.float32)
    xpg_ref[...] = y[: 2 * D_HID, :].T
    xr_ref[...] = y[2 * D_HID:, :].T


def _tc_mid_body(a0, a1, d0, d1, xr, b1r, w2, hp_ref, hr_ref):
    deg = jnp.maximum(d0[...] + d1[...], 1.0)
    h = (a0[...] + a1[...]) / deg + xr[...] + b1r[...]
    h = jnp.where(h > 0, h, jnp.exp(h) - 1.0)
    y = jnp.dot(h, w2[...], preferred_element_type=jnp.float32)
    hp_ref[...] = y[:, :D_HID]
    hr_ref[...] = y[:, D_HID:D_HID + D_OUT]


def _tc_out_body(b0, b1, d0, d1, hr, b2r, o_ref):
    deg = jnp.maximum(d0[...] + d1[...], 1.0)
    sc = (b0[...] + b1[...])[:, :D_OUT] / deg + hr[...] + b2r[...]
    m = jnp.max(sc, axis=1, keepdims=True)
    ex = jnp.exp(sc - m)
    o_ref[...] = (sc - m) - jnp.log(jnp.sum(ex, axis=1, keepdims=True))


def kernel(x, edge_index, pseudo, W1, root1, bias1, W2, root2, bias2):
    f32 = jnp.float32
    wcat = jnp.concatenate([W1[0], W1[1] - W1[0], root1], axis=1)
    w2all = jnp.zeros((D_HID, 24), f32)
    w2all = (w2all.at[:, 0:D_OUT].set(W2[0])
                  .at[:, 8:8 + D_OUT].set(W2[1] - W2[0])
                  .at[:, 16:16 + D_OUT].set(root2))

    src = edge_index[0]
    dst = edge_index[1]
    u = pseudo[:, 0]
    pad = EPAD - E
    srcp = jnp.concatenate([src, jnp.zeros((pad,), jnp.int32)]).reshape(
        NW * NBLK, EB)
    dstp = jnp.concatenate([dst, jnp.full((pad,), N, jnp.int32)]).reshape(
        NW * NBLK, EB)
    up = jnp.concatenate([u, jnp.zeros((pad,), f32)]).reshape(NW * NBLK, EB)
    z2d = jnp.zeros((RPT, D_HID), f32)
    z1d = jnp.zeros((RPT,), f32)

    xpg, xr = pl.pallas_call(
        _tc_pack1_body,
        grid=((N + BNT - 1) // BNT,),
        in_specs=[pl.BlockSpec((48, D_IN), lambda i: (0, 0)),
                  pl.BlockSpec((D_IN, BNT), lambda i: (0, i))],
        out_specs=[pl.BlockSpec((BNT, 2 * D_HID), lambda i: (i, 0)),
                   pl.BlockSpec((BNT, D_HID), lambda i: (i, 0))],
        out_shape=[jax.ShapeDtypeStruct((N, 2 * D_HID), f32),
                   jax.ShapeDtypeStruct((N, D_HID), f32)],
    )(wcat.T, x.T)

    agg1, deg = _sc_layer1(xpg, srcp, dstp, up, z2d, z1d)
    d0 = deg[0].reshape(NPAD, 1)
    d1 = deg[1].reshape(NPAD, 1)

    hp, hr = pl.pallas_call(
        _tc_mid_body,
        grid=(N // BN,),
        in_specs=[pl.BlockSpec((BN, D_HID), lambda i: (i, 0)),
                  pl.BlockSpec((BN, D_HID), lambda i: (i, 0)),
                  pl.BlockSpec((BN, 1), lambda i: (i, 0)),
                  pl.BlockSpec((BN, 1), lambda i: (i, 0)),
                  pl.BlockSpec((BN, D_HID), lambda i: (i, 0)),
                  pl.BlockSpec((1, D_HID), lambda i: (0, 0)),
                  pl.BlockSpec((D_HID, 24), lambda i: (0, 0))],
        out_specs=[pl.BlockSpec((BN, D_HID), lambda i: (i, 0)),
                   pl.BlockSpec((BN, D_OUT), lambda i: (i, 0))],
        out_shape=[jax.ShapeDtypeStruct((N, D_HID), f32),
                   jax.ShapeDtypeStruct((N, D_OUT), f32)],
    )(agg1[0], agg1[1], d0, d1, xr, bias1.reshape(1, D_HID), w2all)

    agg2 = _sc_layer2(hp, srcp, dstp, up, z2d)

    out = pl.pallas_call(
        _tc_out_body,
        grid=(N // BN,),
        in_specs=[pl.BlockSpec((BN, D_HID), lambda i: (i, 0)),
                  pl.BlockSpec((BN, D_HID), lambda i: (i, 0)),
                  pl.BlockSpec((BN, 1), lambda i: (i, 0)),
                  pl.BlockSpec((BN, 1), lambda i: (i, 0)),
                  pl.BlockSpec((BN, D_OUT), lambda i: (i, 0)),
                  pl.BlockSpec((1, D_OUT), lambda i: (0, 0))],
        out_specs=pl.BlockSpec((BN, D_OUT), lambda i: (i, 0)),
        out_shape=jax.ShapeDtypeStruct((N, D_OUT), f32),
    )(agg2[0], agg2[1], d0, d1, hr, bias2.reshape(1, D_OUT))
    return out


# speedup vs baseline: 1.5287x; 1.1572x over previous
import functools

import jax
import jax.numpy as jnp
from jax import lax
from jax.experimental import pallas as pl
from jax.experimental.pallas import tpu as pltpu
from jax.experimental.pallas import tpu_sc as plsc

N = 10000
E = 640000
D_IN = 1433
D_HID = 16
D_OUT = 7

NC = 2
NS = 16
NW = NC * NS
EB = 128
NBLK = 158
NBUF = 2
EPT = EB * NBLK
EPAD = EPT * NW
NPAD = 10240
RPT = NPAD // NS

BN = 400

_DNUMS = lax.GatherDimensionNumbers(
    offset_dims=(), collapsed_slice_dims=(0,), start_index_map=(0,))


def _bcast_lane(vec, t):
    ix = jnp.full((16, 1), t, jnp.int32)
    return lax.gather(vec, ix, _DNUMS, (1,),
                      mode=lax.GatherScatterMode.PROMISE_IN_BOUNDS)


def _sc_mesh():
    return plsc.VectorSubcoreMesh(core_axis_name="c", subcore_axis_name="s")


@functools.partial(
    pl.kernel,
    mesh=_sc_mesh(),
    compiler_params=pltpu.CompilerParams(use_tc_tiling_on_sc=False),
    out_type=[
        jax.ShapeDtypeStruct((NC, NPAD, D_HID), jnp.float32),
        jax.ShapeDtypeStruct((NC, NPAD), jnp.float32),
    ],
    scratch_types=[
        pltpu.VMEM((NBLK, EB), jnp.int32),
        pltpu.VMEM((NBLK, EB), jnp.int32),
        pltpu.VMEM((NBLK, EB), jnp.float32),
        pltpu.VMEM((EB,), jnp.float32),
        pltpu.VMEM((NBUF, EB, D_HID), jnp.int32),
        pltpu.VMEM((NBUF, EB, D_HID), jnp.float32),
        pltpu.VMEM((RPT, D_HID), jnp.float32),
        pltpu.VMEM((RPT,), jnp.float32),
        pltpu.VMEM_SHARED((NPAD, D_HID), jnp.float32),
        pltpu.VMEM_SHARED((NPAD,), jnp.float32),
        pltpu.SemaphoreType.DMA((NBUF,)),
        pltpu.SemaphoreType.DMA((NBUF,)),
        pltpu.SemaphoreType.DMA((NBUF,)),
    ],
)
def _sc_layer1(xp, srcp, dstp, up, z2d, z1d, agg_out, deg_out,
               srca, dsta, ua, onesv, rows, msg, zbuf, zvec,
               aggsh, degsh, gsem, ssem, dsem):
    c = lax.axis_index("c")
    s = lax.axis_index("s")
    wid = s * NC + c

    pltpu.sync_copy(z2d, zbuf)
    pltpu.sync_copy(z1d, zvec)
    pltpu.sync_copy(zbuf, aggsh.at[pl.ds(s * RPT, RPT)])
    pltpu.sync_copy(zvec, degsh.at[pl.ds(s * RPT, RPT)])
    for i in range(EB // 16):
        onesv[pl.ds(i * 16, 16)] = jnp.ones((16,), jnp.float32)

    pltpu.sync_copy(srcp.at[pl.ds(wid * NBLK, NBLK)], srca)
    pltpu.sync_copy(dstp.at[pl.ds(wid * NBLK, NBLK)], dsta)
    pltpu.sync_copy(up.at[pl.ds(wid * NBLK, NBLK)], ua)
    plsc.subcore_barrier()

    for b in range(NBUF):
        pltpu.async_copy(xp.at[srca.at[b]], rows.at[b], gsem.at[b])

    def pairblk(jj, carry):
        for b in range(NBUF):
            j = jj * NBUF + b
            pltpu.make_async_copy(xp.at[srca.at[b]], rows.at[b],
                                  gsem.at[b]).wait()

            @pl.when(j >= NBUF)
            def _():
                pltpu.make_async_copy(msg.at[b], aggsh.at[dsta.at[j]],
                                      ssem.at[b]).wait()
                pltpu.make_async_copy(onesv, degsh.at[dsta.at[j]],
                                      dsem.at[b]).wait()

            for g in range(EB // 16):
                base = g * 16
                u16 = ua[j, pl.ds(base, 16)]
                for t in range(16):
                    ub = _bcast_lane(u16, t)
                    e = base + t
                    v = rows[b, e, :]
                    a = lax.bitcast_convert_type(
                        lax.shift_left(v, 16), jnp.float32)
                    bb = lax.bitcast_convert_type(
                        jnp.bitwise_and(v, jnp.int32(-65536)), jnp.float32)
                    msg[b, e, :] = a + ub * bb

            @pl.when(j + NBUF < NBLK)
            def _():
                pltpu.async_copy(xp.at[srca.at[j + NBUF]], rows.at[b],
                                 gsem.at[b])

            pltpu.async_copy(msg.at[b], aggsh.at[dsta.at[j]], ssem.at[b],
                             add=True)
            pltpu.async_copy(onesv, degsh.at[dsta.at[j]], dsem.at[b],
                             add=True)
        return carry

    lax.fori_loop(0, NBLK // NBUF, pairblk, 0)
    for b in range(NBUF):
        pltpu.make_async_copy(msg.at[b], aggsh.at[dsta.at[b]],
                              ssem.at[b]).wait()
        pltpu.make_async_copy(onesv, degsh.at[dsta.at[b]],
                              dsem.at[b]).wait()
    plsc.subcore_barrier()

    pltpu.sync_copy(aggsh.at[pl.ds(s * RPT, RPT)],
                    agg_out.at[c, pl.ds(s * RPT, RPT)])
    pltpu.sync_copy(degsh.at[pl.ds(s * RPT, RPT)],
                    deg_out.at[c, pl.ds(s * RPT, RPT)])


@functools.partial(
    pl.kernel,
    mesh=_sc_mesh(),
    compiler_params=pltpu.CompilerParams(use_tc_tiling_on_sc=False),
    out_type=jax.ShapeDtypeStruct((NC, NPAD, D_HID), jnp.float32),
    scratch_types=[
        pltpu.VMEM((NBLK, EB), jnp.int32),
        pltpu.VMEM((NBLK, EB), jnp.int32),
        pltpu.VMEM((NBLK, EB), jnp.float32),
        pltpu.VMEM((NBUF, EB, D_HID), jnp.float32),
        pltpu.VMEM((NBUF, EB, D_HID), jnp.float32),
        pltpu.VMEM((RPT, D_HID), jnp.float32),
        pltpu.VMEM_SHARED((NPAD, D_HID), jnp.float32),
        pltpu.SemaphoreType.DMA((NBUF,)),
        pltpu.SemaphoreType.DMA((NBUF,)),
    ],
)
def _sc_layer2(hp, srcp, dstp, up, z2d, agg_out,
               srca, dsta, ua, rows, msg, zbuf, aggsh, gsem, ssem):
    c = lax.axis_index("c")
    s = lax.axis_index("s")
    wid = s * NC + c

    pltpu.sync_copy(z2d, zbuf)
    pltpu.sync_copy(zbuf, aggsh.at[pl.ds(s * RPT, RPT)])
    pltpu.sync_copy(srcp.at[pl.ds(wid * NBLK, NBLK)], srca)
    pltpu.sync_copy(dstp.at[pl.ds(wid * NBLK, NBLK)], dsta)
    pltpu.sync_copy(up.at[pl.ds(wid * NBLK, NBLK)], ua)
    plsc.subcore_barrier()

    lanes = lax.iota(jnp.int32, 16)
    hi_sel = lanes < 8
    shift_ix = jnp.bitwise_or(lanes, 8).reshape(16, 1)

    for b in range(NBUF):
        pltpu.async_copy(hp.at[srca.at[b]], rows.at[b], gsem.at[b])

    def pairblk(jj, carry):
        for b in range(NBUF):
            j = jj * NBUF + b
            pltpu.make_async_copy(hp.at[srca.at[b]], rows.at[b],
                                  gsem.at[b]).wait()

            @pl.when(j >= NBUF)
            def _():
                pltpu.make_async_copy(msg.at[b], aggsh.at[dsta.at[j]],
                                      ssem.at[b]).wait()

            for g in range(EB // 16):
                base = g * 16
                u16 = ua[j, pl.ds(base, 16)]
                for t in range(16):
                    ub = _bcast_lane(u16, t)
                    e = base + t
                    v = rows[b, e, :]
                    w = v * jnp.where(hi_sel, jnp.float32(1.0), ub)
                    msg[b, e, :] = w + lax.gather(
                        w, shift_ix, _DNUMS, (1,),
                        mode=lax.GatherScatterMode.PROMISE_IN_BOUNDS)

            @pl.when(j + NBUF < NBLK)
            def _():
                pltpu.async_copy(hp.at[srca.at[j + NBUF]], rows.at[b],
                                 gsem.at[b])

            pltpu.async_copy(msg.at[b], aggsh.at[dsta.at[j]], ssem.at[b],
                             add=True)
        return carry

    lax.fori_loop(0, NBLK // NBUF, pairblk, 0)
    for b in range(NBUF):
        pltpu.make_async_copy(msg.at[b], aggsh.at[dsta.at[b]],
                              ssem.at[b]).wait()
    plsc.subcore_barrier()

    pltpu.sync_copy(aggsh.at[pl.ds(s * RPT, RPT)],
                    agg_out.at[c, pl.ds(s * RPT, RPT)])


BNT = 1280


def _tc_pack1_body(wt_ref, xt_ref, xpg_ref, xr_ref):
    y = jnp.dot(wt_ref[...], xt_ref[...], preferred_element_type=jnp.float32)
    a_bits = lax.bitcast_convert_type(y[:D_HID, :].T, jnp.int32)
    b_bits = lax.bitcast_convert_type(y[D_HID:2 * D_HID, :].T, jnp.int32)
    xpg_ref[...] = jnp.bitwise_or(
        jnp.bitwise_and(b_bits, jnp.int32(-65536)),
        lax.shift_right_logical(a_bits, 16))
    xr_ref[...] = y[2 * D_HID:, :].T


def _tc_mid_body(a0, a1, d0, d1, xr, b1r, w2, hp_ref, hr_ref):
    deg = jnp.maximum(d0[...] + d1[...], 1.0)
    h = (a0[...] + a1[...]) / deg + xr[...] + b1r[...]
    h = jnp.where(h > 0, h, jnp.exp(h) - 1.0)
    y = jnp.dot(h, w2[...], preferred_element_type=jnp.float32)
    hp_ref[...] = y[:, :D_HID]
    hr_ref[...] = y[:, D_HID:D_HID + D_OUT]


def _tc_out_body(b0, b1, d0, d1, hr, b2r, o_ref):
    deg = jnp.maximum(d0[...] + d1[...], 1.0)
    sc = (b0[...] + b1[...])[:, :D_OUT] / deg + hr[...] + b2r[...]
    m = jnp.max(sc, axis=1, keepdims=True)
    ex = jnp.exp(sc - m)
    o_ref[...] = (sc - m) - jnp.log(jnp.sum(ex, axis=1, keepdims=True))


def kernel(x, edge_index, pseudo, W1, root1, bias1, W2, root2, bias2):
    f32 = jnp.float32
    wcat = jnp.concatenate([W1[0], W1[1] - W1[0], root1], axis=1)
    w2all = jnp.zeros((D_HID, 24), f32)
    w2all = (w2all.at[:, 0:D_OUT].set(W2[0])
                  .at[:, 8:8 + D_OUT].set(W2[1] - W2[0])
                  .at[:, 16:16 + D_OUT].set(root2))

    src = edge_index[0]
    dst = edge_index[1]
    u = pseudo[:, 0]
    pad = EPAD - E
    srcp = jnp.concatenate([src, jnp.zeros((pad,), jnp.int32)]).reshape(
        NW * NBLK, EB)
    dstp = jnp.concatenate([dst, jnp.full((pad,), N, jnp.int32)]).reshape(
        NW * NBLK, EB)
    up = jnp.concatenate([u, jnp.zeros((pad,), f32)]).reshape(NW * NBLK, EB)
    z2d = jnp.zeros((RPT, D_HID), f32)
    z1d = jnp.zeros((RPT,), f32)

    xpg, xr = pl.pallas_call(
        _tc_pack1_body,
        grid=((N + BNT - 1) // BNT,),
        in_specs=[pl.BlockSpec((48, D_IN), lambda i: (0, 0)),
                  pl.BlockSpec((D_IN, BNT), lambda i: (0, i))],
        out_specs=[pl.BlockSpec((BNT, D_HID), lambda i: (i, 0)),
                   pl.BlockSpec((BNT, D_HID), lambda i: (i, 0))],
        out_shape=[jax.ShapeDtypeStruct((N, D_HID), jnp.int32),
                   jax.ShapeDtypeStruct((N, D_HID), f32)],
    )(wcat.T, x.T)

    agg1, deg = _sc_layer1(xpg, srcp, dstp, up, z2d, z1d)
    d0 = deg[0].reshape(NPAD, 1)
    d1 = deg[1].reshape(NPAD, 1)

    hp, hr = pl.pallas_call(
        _tc_mid_body,
        grid=(N // BN,),
        in_specs=[pl.BlockSpec((BN, D_HID), lambda i: (i, 0)),
                  pl.BlockSpec((BN, D_HID), lambda i: (i, 0)),
                  pl.BlockSpec((BN, 1), lambda i: (i, 0)),
                  pl.BlockSpec((BN, 1), lambda i: (i, 0)),
                  pl.BlockSpec((BN, D_HID), lambda i: (i, 0)),
                  pl.BlockSpec((1, D_HID), lambda i: (0, 0)),
                  pl.BlockSpec((D_HID, 24), lambda i: (0, 0))],
        out_specs=[pl.BlockSpec((BN, D_HID), lambda i: (i, 0)),
                   pl.BlockSpec((BN, D_OUT), lambda i: (i, 0))],
        out_shape=[jax.ShapeDtypeStruct((N, D_HID), f32),
                   jax.ShapeDtypeStruct((N, D_OUT), f32)],
    )(agg1[0], agg1[1], d0, d1, xr, bias1.reshape(1, D_HID), w2all)

    agg2 = _sc_layer2(hp, srcp, dstp, up, z2d)

    out = pl.pallas_call(
        _tc_out_body,
        grid=(N // BN,),
        in_specs=[pl.BlockSpec((BN, D_HID), lambda i: (i, 0)),
                  pl.BlockSpec((BN, D_HID), lambda i: (i, 0)),
                  pl.BlockSpec((BN, 1), lambda i: (i, 0)),
                  pl.BlockSpec((BN, 1), lambda i: (i, 0)),
                  pl.BlockSpec((BN, D_OUT), lambda i: (i, 0)),
                  pl.BlockSpec((1, D_OUT), lambda i: (0, 0))],
        out_specs=pl.BlockSpec((BN, D_OUT), lambda i: (i, 0)),
        out_shape=jax.ShapeDtypeStruct((N, D_OUT), f32),
    )(agg2[0], agg2[1], d0, d1, hr, bias2.reshape(1, D_OUT))
    return out
